# baseline (speedup 1.0000x reference)
import jax
import jax.numpy as jnp
from jax.experimental import pallas as pl
from jax.experimental.pallas import tpu as pltpu

E_BLK = 2560
R_BLK = 2560


def _node_table_kernel(tf_ref, wc_ref, bc_ref, out_ref):
    out_ref[...] = jnp.dot(tf_ref[...], wc_ref[...],
                           preferred_element_type=jnp.float32) + bc_ref[...]


def _edge_kernel(p, fdg_ref, rij_ref, se_ref, hg_ref, wv_ref, cv_ref,
                 asv_ref, adv_ref, *refs):
    if p == 0:
        s2_ref, den_ref, de_ref = refs
    else:
        s2_ref, de_ref = refs
        den_ref = None
    step = pl.program_id(0)

    @pl.when(step == 0)
    def _init_s2():
        s2_ref[...] = jnp.zeros_like(s2_ref)
        if den_ref is not None:
            den_ref[...] = jnp.zeros_like(den_ref)

    de_ref[...] = jnp.dot(fdg_ref[...], wv_ref[...],
                          preferred_element_type=jnp.float32) + cv_ref[0, 0]

    lane = jax.lax.broadcasted_iota(jnp.int32, (1, 128), 1)
    asv = asv_ref[...]
    adv = adv_ref[...]

    def body(j, carry):
        src = rij_ref[0, j]
        dst = rij_ref[1, j]
        isrc = (se_ref[src // 2] >> ((src % 2) * 16)) & 0xFFFF
        idst = (se_ref[dst // 2] >> ((dst % 2) * 16)) & 0xFFFF
        hs = hg_ref[pl.ds(isrc, 1), :]
        hd = hg_ref[pl.ds(idst, 1), :]
        e = (jnp.sum(hs * asv, axis=1, keepdims=True)
             + jnp.sum(hd * adv, axis=1, keepdims=True)
             + de_ref[pl.ds(j, 1), :])
        e = jnp.where(e >= 0.0, e, 0.2 * e)
        ex = jnp.exp(e)

        q = dst // 2 - p * s2_ref.shape[0]

        @pl.when((q >= 0) & (q < s2_ref.shape[0]))
        def _acc_s2():
            payload = ex * hs
            wide = jnp.concatenate([payload, payload], axis=1)
            lo = (dst % 2) * 64
            m = (lane >= lo) & (lane < lo + 64)
            s2_ref[pl.ds(q, 1), :] = (s2_ref[pl.ds(q, 1), :]
                                      + jnp.where(m, wide, 0.0))

        if den_ref is not None:
            dr = dst // 128
            dl = dst % 128
            den_ref[pl.ds(dr, 1), :] = (den_ref[pl.ds(dr, 1), :]
                                        + jnp.where(lane == dl, ex, 0.0))
        return carry

    jax.lax.fori_loop(0, E_BLK, body, 0)


def _pool_kernel(s2_ref, den_ref, bse_ref, bso_ref, wout_ref, bout_ref,
                 pooled_ref, out_ref):
    step = pl.program_id(0)
    nstep = pl.num_programs(0)

    @pl.when(step == 0)
    def _init():
        pooled_ref[...] = jnp.zeros_like(pooled_ref)
        out_ref[...] = jnp.zeros_like(out_ref)

    half = R_BLK // 2
    q = jax.lax.broadcasted_iota(jnp.int32, (half, 1), 0)
    lane = jax.lax.broadcasted_iota(jnp.int32, (half, 128), 1)

    rowsel = (jax.lax.broadcasted_iota(jnp.int32, (half, 2500), 1)
              == step * 20 + q // 64).astype(jnp.float32)
    dexp = jnp.dot(rowsel, den_ref[...],
                   preferred_element_type=jnp.float32)
    le = 2 * (q % 64)
    den_e = jnp.sum(jnp.where(lane == le, dexp, 0.0), axis=1, keepdims=True)
    den_o = jnp.sum(jnp.where(lane == le + 1, dexp, 0.0), axis=1, keepdims=True)

    s2 = s2_ref[...]
    he = s2[:, 0:64] / (den_e + 1e-16)
    ho = s2[:, 64:128] / (den_o + 1e-16)
    he = jnp.where(he > 0.0, he, jnp.exp(jnp.minimum(he, 0.0)) - 1.0)
    ho = jnp.where(ho > 0.0, ho, jnp.exp(jnp.minimum(ho, 0.0)) - 1.0)

    ge = bse_ref[...].reshape(half, 1)
    go = bso_ref[...].reshape(half, 1)
    gid = jax.lax.broadcasted_iota(jnp.int32, (half, 1024), 1)
    ohe = (gid == ge).astype(jnp.float32)
    oho = (gid == go).astype(jnp.float32)
    contrib = (jax.lax.dot_general(ohe, he, (((0,), (0,)), ((), ())),
                                   preferred_element_type=jnp.float32)
               + jax.lax.dot_general(oho, ho, (((0,), (0,)), ((), ())),
                                     preferred_element_type=jnp.float32))
    pooled_ref[...] += contrib

    @pl.when(step == nstep - 1)
    def _fin():
        logits = jnp.dot(pooled_ref[...], wout_ref[...],
                         preferred_element_type=jnp.float32) + bout_ref[...]
        mx = jnp.max(logits, axis=1, keepdims=True)
        z = jnp.exp(logits - mx)
        out_ref[...] = z / jnp.sum(z, axis=1, keepdims=True)


def kernel(target_features, feature_dist_graph, affinities, W_emb, b_emb,
           W_dist, b_dist, W_gat, att_src, att_dst, att_edge, W_out, b_out,
           start_end_env, rij_dist_pairs, b_scope, l_scope, names):
    f32 = jnp.float32
    E = feature_dist_graph.shape[0]
    M = start_end_env.shape[0] * 2

    wc = jnp.dot(W_emb, W_gat)
    bc = jnp.dot(b_emb, W_gat).reshape(1, -1)
    wv = jnp.dot(W_dist, att_edge).reshape(-1, 1)
    cv = jnp.dot(b_dist, att_edge).reshape(1, 1)

    hg = pl.pallas_call(
        _node_table_kernel,
        grid=(5,),
        in_specs=[pl.BlockSpec((2000, 128), lambda i: (i, 0)),
                  pl.BlockSpec((128, 64), lambda i: (0, 0)),
                  pl.BlockSpec((1, 64), lambda i: (0, 0))],
        out_specs=pl.BlockSpec((2000, 64), lambda i: (i, 0)),
        out_shape=jax.ShapeDtypeStruct((10000, 64), f32),
    )(target_features, wc, bc)
    hg_t = jnp.concatenate(
        [jnp.zeros((1, 64), f32), hg, jnp.zeros((7, 64), f32)], axis=0)

    se = start_end_env.reshape(-1)
    se_p = se[0::2] | (se[1::2] << 16)
    rij_t = rij_dist_pairs.T
    asv = att_src.reshape(1, -1).astype(f32)
    adv = att_dst.reshape(1, -1).astype(f32)

    edge_in_specs = [
        pl.BlockSpec((E_BLK, 128), lambda i: (i, 0)),
        pl.BlockSpec((2, E_BLK), lambda i: (0, i), memory_space=pltpu.SMEM),
        pl.BlockSpec(memory_space=pltpu.SMEM),
        pl.BlockSpec((10008, 64), lambda i: (0, 0)),
        pl.BlockSpec((128, 1), lambda i: (0, 0)),
        pl.BlockSpec((1, 1), lambda i: (0, 0), memory_space=pltpu.SMEM),
        pl.BlockSpec((1, 64), lambda i: (0, 0)),
        pl.BlockSpec((1, 64), lambda i: (0, 0)),
    ]
    edge_args = (feature_dist_graph, rij_t, se_p, hg_t, wv, cv, asv, adv)

    s2a, den = pl.pallas_call(
        lambda *r: _edge_kernel(0, *r),
        grid=(E // E_BLK,),
        in_specs=edge_in_specs,
        out_specs=[pl.BlockSpec((M // 4, 128), lambda i: (0, 0)),
                   pl.BlockSpec((M // 128, 128), lambda i: (0, 0))],
        out_shape=[jax.ShapeDtypeStruct((M // 4, 128), f32),
                   jax.ShapeDtypeStruct((M // 128, 128), f32)],
        scratch_shapes=[pltpu.VMEM((E_BLK, 1), f32)],
    )(*edge_args)
    s2b = pl.pallas_call(
        lambda *r: _edge_kernel(1, *r),
        grid=(E // E_BLK,),
        in_specs=edge_in_specs,
        out_specs=pl.BlockSpec((M // 4, 128), lambda i: (0, 0)),
        out_shape=jax.ShapeDtypeStruct((M // 4, 128), f32),
        scratch_shapes=[pltpu.VMEM((E_BLK, 1), f32)],
    )(*edge_args)
    s2 = jnp.concatenate([s2a, s2b], axis=0)

    nblk = M // R_BLK
    bse = b_scope[0::2].reshape(nblk, R_BLK // 2, 1)
    bso = b_scope[1::2].reshape(nblk, R_BLK // 2, 1)

    _, out = pl.pallas_call(
        _pool_kernel,
        grid=(nblk,),
        in_specs=[
            pl.BlockSpec((R_BLK // 2, 128), lambda i: (i, 0)),
            pl.BlockSpec((M // 128, 128), lambda i: (0, 0)),
            pl.BlockSpec((1, R_BLK // 2, 1), lambda i: (i, 0, 0)),
            pl.BlockSpec((1, R_BLK // 2, 1), lambda i: (i, 0, 0)),
            pl.BlockSpec((64, 2), lambda i: (0, 0)),
            pl.BlockSpec((1, 2), lambda i: (0, 0)),
        ],
        out_specs=[pl.BlockSpec((1024, 64), lambda i: (0, 0)),
                   pl.BlockSpec((1024, 2), lambda i: (0, 0))],
        out_shape=[jax.ShapeDtypeStruct((1024, 64), f32),
                   jax.ShapeDtypeStruct((1024, 2), f32)],
    )(s2, den, bse, bso, W_out.astype(f32), b_out.reshape(1, 2))
    return out
